# baseline (speedup 1.0000x reference)
import jax
import jax.numpy as jnp
from jax.experimental import pallas as pl
from jax.experimental.pallas import tpu as pltpu

_NUM_OUTPUTS = 1024
_NUM_INPUTS = 100000
_HALF = _NUM_OUTPUTS * _NUM_INPUTS

_K1 = (1832780943, 270669613)
_K2 = (64467757, 2916123636)

_ROWS = 8
_SUB = 512
_T = 196
_U = 32
_CPAD = _SUB * _T

_TINY = 1.1754943508222875e-38


def _threefry2x32(key, j):
    k0 = jnp.uint32(key[0])
    k1 = jnp.uint32(key[1])
    ks = (k0, k1, k0 ^ k1 ^ jnp.uint32(0x1BD11BDA))
    rot = ((13, 15, 26, 6), (17, 29, 16, 24))
    x1 = j + ks[1]
    x0 = j + (ks[0] + ks[1])
    t = (x1 << 13) | (x1 >> 19)
    x1 = x0 ^ t
    for r in (15, 26, 6):
        x0 = x0 + x1
        x1 = (x1 << r) | (x1 >> (32 - r))
        x1 = x0 ^ x1
    x0 = x0 + ks[1]
    x1 = x1 + (ks[2] + jnp.uint32(1))
    for i in range(1, 5):
        for r in rot[i % 2]:
            x0 = x0 + x1
            x1 = (x1 << r) | (x1 >> (32 - r))
            x1 = x0 ^ x1
        x0 = x0 + ks[(i + 1) % 3]
        x1 = x1 + (ks[(i + 2) % 3] + jnp.uint32(i + 1))
    return x0, x1


def _bits(key, j):
    b0, b1 = _threefry2x32(key, j)
    return b0 ^ b1


def _unit_float(bits):
    fb = (bits >> jnp.uint32(9)) | jnp.uint32(0x3F800000)
    return jax.lax.bitcast_convert_type(fb, jnp.float32) - jnp.float32(1.0)


def _gumbel(bits):
    f = _unit_float(bits)
    u = f + jnp.float32(_TINY)
    return -jnp.log(-jnp.log(u))


def _kern(x_ref, p_ref, samp_ref, nots_ref):
    r = pl.program_id(0)

    row_l = jax.lax.broadcasted_iota(jnp.uint32, (_ROWS, _SUB), 0)
    col_l = jax.lax.broadcasted_iota(jnp.int32, (_ROWS, _SUB), 1)
    j_base = (row_l + (r * _ROWS).astype(jnp.uint32)) * jnp.uint32(
        _NUM_INPUTS
    ) + col_l.astype(jnp.uint32)

    neg_inf = jnp.float32(-jnp.inf)
    int_max = jnp.int32(2**31 - 1)

    def step(t, carry, masked):
        av0, ai0, av1, ai1 = carry
        off = t * _SUB
        x = x_ref[:, pl.ds(off, _SUB)]
        j = j_base + off.astype(jnp.uint32)

        v0 = x + _gumbel(_bits(_K1, j))
        if masked:
            v0 = jnp.where(col_l + off < _NUM_INPUTS, v0, neg_inf)
        u0 = v0 > av0
        av0 = jnp.where(u0, v0, av0)
        ai0 = jnp.where(u0, t, ai0)

        v1 = x + _gumbel(_bits(_K1, j + jnp.uint32(_HALF)))
        if masked:
            v1 = jnp.where(col_l + off < _NUM_INPUTS, v1, neg_inf)
        u1 = v1 > av1
        av1 = jnp.where(u1, v1, av1)
        ai1 = jnp.where(u1, t, ai1)
        return av0, ai0, av1, ai1

    init = (
        jnp.full((_ROWS, _SUB), neg_inf, jnp.float32),
        jnp.full((_ROWS, _SUB), int_max, jnp.int32),
        jnp.full((_ROWS, _SUB), neg_inf, jnp.float32),
        jnp.full((_ROWS, _SUB), int_max, jnp.int32),
    )
    hot = _T - 1

    def bodyu(t, c):
        for k in range(_U):
            c = step(_U * t + k, c, False)
        return c

    carry = jax.lax.fori_loop(0, hot // _U, bodyu, init)
    for k in range(hot - hot % _U, hot):
        carry = step(jnp.int32(k), carry, False)
    av0, ai0, av1, ai1 = step(jnp.int32(_T - 1), carry, True)

    finals = []
    for av, ai in ((av0, ai0), (av1, ai1)):
        m = jnp.max(av, axis=1, keepdims=True)
        gi = ai * _SUB + col_l
        li = jnp.min(jnp.where(av == m, gi, int_max), axis=1, keepdims=True)
        finals.append(li)
    samp_ref[...] = jnp.concatenate(finals, axis=1)

    o = jax.lax.broadcasted_iota(jnp.uint32, (_ROWS, 1), 0) + (
        r * _ROWS
    ).astype(jnp.uint32)
    f = _unit_float(_bits(_K2, o))
    u = jnp.maximum(f * jnp.float32(1.0) + jnp.float32(0.0), jnp.float32(0.0))
    nots_ref[...] = (u < p_ref[...]).astype(jnp.float32)


def kernel(adjacency_probability_matrix, not_probability):
    nr = _NUM_OUTPUTS // _ROWS
    samples, nots = pl.pallas_call(
        _kern,
        grid=(nr,),
        compiler_params=pltpu.CompilerParams(
            dimension_semantics=("parallel",)
        ),
        in_specs=[
            pl.BlockSpec((_ROWS, _CPAD), lambda r: (r, 0)),
            pl.BlockSpec((_ROWS, 1), lambda r: (r, 0)),
        ],
        out_specs=[
            pl.BlockSpec((_ROWS, 2), lambda r: (r, 0)),
            pl.BlockSpec((_ROWS, 1), lambda r: (r, 0)),
        ],
        out_shape=[
            jax.ShapeDtypeStruct((_NUM_OUTPUTS, 2), jnp.int32),
            jax.ShapeDtypeStruct((_NUM_OUTPUTS, 1), jnp.float32),
        ],
    )(adjacency_probability_matrix, not_probability.reshape(_NUM_OUTPUTS, 1))
    return samples, nots.reshape(_NUM_OUTPUTS)


# speedup vs baseline: 1.0090x; 1.0090x over previous
import jax
import jax.numpy as jnp
from jax.experimental import pallas as pl
from jax.experimental.pallas import tpu as pltpu

_NUM_OUTPUTS = 1024
_NUM_INPUTS = 100000
_HALF = _NUM_OUTPUTS * _NUM_INPUTS

_K1 = (1832780943, 270669613)
_K2 = (64467757, 2916123636)

_ROWS = 8
_SUB = 512
_T = 196
_U = 32
_CPAD = _SUB * _T

_TINY = 1.1754943508222875e-38


def _ks(key):
    k0 = jnp.uint32(key[0])
    k1 = jnp.uint32(key[1])
    return (k0, k1, k0 ^ k1 ^ jnp.uint32(0x1BD11BDA))


def _threefry_tail(ks, x0, x1):
    rot = ((13, 15, 26, 6), (17, 29, 16, 24))
    t = (x1 << 13) | (x1 >> 19)
    x1 = x0 ^ t
    for r in (15, 26, 6):
        x0 = x0 + x1
        x1 = (x1 << r) | (x1 >> (32 - r))
        x1 = x0 ^ x1
    x0 = x0 + ks[1]
    x1 = x1 + (ks[2] + jnp.uint32(1))
    for i in range(1, 5):
        for r in rot[i % 2]:
            x0 = x0 + x1
            x1 = (x1 << r) | (x1 >> (32 - r))
            x1 = x0 ^ x1
        x0 = x0 + ks[(i + 1) % 3]
        x1 = x1 + (ks[(i + 2) % 3] + jnp.uint32(i + 1))
    return x0, x1


def _bits_from(ks, x0, x1):
    b0, b1 = _threefry_tail(ks, x0, x1)
    return b0 ^ b1


def _bits(key, j):
    ks = _ks(key)
    return _bits_from(ks, j + (ks[0] + ks[1]), j + ks[1])


def _unit_float(bits):
    fb = (bits >> jnp.uint32(9)) | jnp.uint32(0x3F800000)
    return jax.lax.bitcast_convert_type(fb, jnp.float32) - jnp.float32(1.0)


def _gumbel(bits):
    f = _unit_float(bits)
    u = f + jnp.float32(_TINY)
    return -jnp.log(-jnp.log(u))


def _kern(x_ref, p_ref, samp_ref, nots_ref):
    r = pl.program_id(0)

    row_l = jax.lax.broadcasted_iota(jnp.uint32, (_ROWS, _SUB), 0)
    col_l = jax.lax.broadcasted_iota(jnp.int32, (_ROWS, _SUB), 1)
    j_base = (row_l + (r * _ROWS).astype(jnp.uint32)) * jnp.uint32(
        _NUM_INPUTS
    ) + col_l.astype(jnp.uint32)

    neg_inf = jnp.float32(-jnp.inf)
    int_max = jnp.int32(2**31 - 1)

    ksv = _ks(_K1)
    jb1 = j_base + ksv[1]
    jb0 = j_base + (ksv[0] + ksv[1])

    def step(t, carry, masked):
        av0, ai0, av1, ai1 = carry
        off = t * _SUB
        x = x_ref[:, pl.ds(off, _SUB)]
        offu = jnp.uint32(off) if isinstance(off, int) else off.astype(jnp.uint32)
        offh = offu + jnp.uint32(_HALF)

        v0 = x + _gumbel(_bits_from(ksv, jb0 + offu, jb1 + offu))
        if masked:
            v0 = jnp.where(col_l + off < _NUM_INPUTS, v0, neg_inf)
        u0 = v0 > av0
        av0 = jnp.where(u0, v0, av0)
        ai0 = jnp.where(u0, t, ai0)

        v1 = x + _gumbel(_bits_from(ksv, jb0 + offh, jb1 + offh))
        if masked:
            v1 = jnp.where(col_l + off < _NUM_INPUTS, v1, neg_inf)
        u1 = v1 > av1
        av1 = jnp.where(u1, v1, av1)
        ai1 = jnp.where(u1, t, ai1)
        return av0, ai0, av1, ai1

    init = (
        jnp.full((_ROWS, _SUB), neg_inf, jnp.float32),
        jnp.full((_ROWS, _SUB), int_max, jnp.int32),
        jnp.full((_ROWS, _SUB), neg_inf, jnp.float32),
        jnp.full((_ROWS, _SUB), int_max, jnp.int32),
    )
    hot = _T - 1

    def bodyu(t, c):
        for k in range(_U):
            c = step(_U * t + k, c, False)
        return c

    carry = jax.lax.fori_loop(0, hot // _U, bodyu, init)
    for k in range(hot - hot % _U, hot):
        carry = step(jnp.int32(k), carry, False)
    av0, ai0, av1, ai1 = step(jnp.int32(_T - 1), carry, True)

    finals = []
    for av, ai in ((av0, ai0), (av1, ai1)):
        m = jnp.max(av, axis=1, keepdims=True)
        gi = ai * _SUB + col_l
        li = jnp.min(jnp.where(av == m, gi, int_max), axis=1, keepdims=True)
        finals.append(li)
    samp_ref[...] = jnp.concatenate(finals, axis=1)

    o = jax.lax.broadcasted_iota(jnp.uint32, (_ROWS, 1), 0) + (
        r * _ROWS
    ).astype(jnp.uint32)
    f = _unit_float(_bits(_K2, o))
    u = jnp.maximum(f * jnp.float32(1.0) + jnp.float32(0.0), jnp.float32(0.0))
    nots_ref[...] = (u < p_ref[...]).astype(jnp.float32)


def kernel(adjacency_probability_matrix, not_probability):
    nr = _NUM_OUTPUTS // _ROWS
    samples, nots = pl.pallas_call(
        _kern,
        grid=(nr,),
        compiler_params=pltpu.CompilerParams(
            dimension_semantics=("parallel",)
        ),
        in_specs=[
            pl.BlockSpec((_ROWS, _CPAD), lambda r: (r, 0)),
            pl.BlockSpec((_ROWS, 1), lambda r: (r, 0)),
        ],
        out_specs=[
            pl.BlockSpec((_ROWS, 2), lambda r: (r, 0)),
            pl.BlockSpec((_ROWS, 1), lambda r: (r, 0)),
        ],
        out_shape=[
            jax.ShapeDtypeStruct((_NUM_OUTPUTS, 2), jnp.int32),
            jax.ShapeDtypeStruct((_NUM_OUTPUTS, 1), jnp.float32),
        ],
    )(adjacency_probability_matrix, not_probability.reshape(_NUM_OUTPUTS, 1))
    return samples, nots.reshape(_NUM_OUTPUTS)
